# baseline (speedup 1.0000x reference)
import functools

import jax
import jax.numpy as jnp
from jax import lax
from jax.experimental import pallas as pl
from jax.experimental.pallas import tpu as pltpu

_HIDDEN = 1000
_BATCH = 4096
_SEQ = 26
_BB = 128
_S = 4
_SB = _BB // _S
_G = _BATCH // _BB


def _onehot_block(x_ref, o_hbm, buf, sems):
    i = pl.program_id(0)
    b = i % 2

    @pl.when(i >= 2)
    def _():
        for s in range(_S):
            pltpu.make_async_copy(
                buf.at[b, pl.ds(s * _SB, _SB)],
                o_hbm.at[pl.ds((i - 2) * _BB + s * _SB, _SB)],
                sems.at[b, s],
            ).wait()

    idx = x_ref[...]
    classes = lax.broadcasted_iota(jnp.int32, (_BB, _SEQ, _HIDDEN), 2)
    buf[b] = (classes == idx[:, :, None]).astype(jnp.float32)

    for s in range(_S):
        pltpu.make_async_copy(
            buf.at[b, pl.ds(s * _SB, _SB)],
            o_hbm.at[pl.ds(i * _BB + s * _SB, _SB)],
            sems.at[b, s],
        ).start()

    @pl.when(i == _G - 1)
    def _():
        for bb in range(2):
            jj = _G - 2 + ((_G + bb) % 2)
            for s in range(_S):
                pltpu.make_async_copy(
                    buf.at[bb, pl.ds(s * _SB, _SB)],
                    o_hbm.at[pl.ds(jj * _BB + s * _SB, _SB)],
                    sems.at[bb, s],
                ).wait()


@jax.jit
def kernel(x):
    return pl.pallas_call(
        _onehot_block,
        grid=(_G,),
        in_specs=[pl.BlockSpec((_BB, _SEQ), lambda i: (i, 0))],
        out_specs=pl.BlockSpec(memory_space=pl.ANY),
        out_shape=jax.ShapeDtypeStruct((_BATCH, _SEQ, _HIDDEN), jnp.float32),
        scratch_shapes=[
            pltpu.VMEM((2, _BB, _SEQ, _HIDDEN), jnp.float32),
            pltpu.SemaphoreType.DMA((2, _S)),
        ],
        compiler_params=pltpu.CompilerParams(
            dimension_semantics=("arbitrary",),
        ),
    )(x.astype(jnp.int32))
